# baseline (speedup 1.0000x reference)
import jax
import jax.numpy as jnp
from jax import lax
from jax.experimental import pallas as pl
from jax.experimental.pallas import tpu as pltpu
from jax.experimental.pallas import tpu_sc as plsc

_IT = 4000
_NW = 16
_L = 16
_N = 100000
_CHUNK = 6272
_LAST = _N - 15 * _CHUNK


def _tree(vals, op):
    while len(vals) > 1:
        half = [op(vals[2 * i], vals[2 * i + 1]) for i in range(len(vals) // 2)]
        if len(vals) % 2:
            half.append(vals[-1])
        vals = half
    return vals[0]


def _sc_softmax_body(alf_hbm, gf_hbm, p_hbm, buf, stage_v, allv, shared, sem):
    s = lax.axis_index("s")
    base = s * _CHUNK
    neg_inf = jnp.full((_L,), -jnp.inf, jnp.float32)

    @pl.when(s < 15)
    def _():
        pltpu.make_async_copy(gf_hbm.at[pl.ds(base, _CHUNK)], buf, sem).start()

    @pl.when(s == 15)
    def _():
        pltpu.make_async_copy(
            gf_hbm.at[pl.ds(base, _LAST)], buf.at[pl.ds(0, _LAST)], sem).start()

    pltpu.sync_copy(alf_hbm, stage_v)
    alfv = stage_v[...]
    a16 = 1.0 / (1.0 + jnp.exp(-alfv))

    @pl.when(s < 15)
    def _():
        pltpu.make_async_copy(gf_hbm.at[pl.ds(base, _CHUNK)], buf, sem).wait()

    @pl.when(s == 15)
    def _():
        pltpu.make_async_copy(
            gf_hbm.at[pl.ds(base, _LAST)], buf.at[pl.ds(0, _LAST)], sem).wait()

    def max_pass(nch, unroll):
        def body(k, m):
            vals = [buf[pl.ds((k * unroll + j) * _L, _L)] for j in range(unroll)]
            return jnp.maximum(m, _tree(vals, jnp.maximum))

        stage_v[...] = lax.fori_loop(0, nch // unroll, body, neg_inf)

    @pl.when(s < 15)
    def _():
        max_pass(_CHUNK // _L, 8)

    @pl.when(s == 15)
    def _():
        max_pass(_LAST // _L, 2)

    pltpu.sync_copy(stage_v, shared.at[pl.ds(s * _L, _L)])
    plsc.subcore_barrier()
    pltpu.sync_copy(shared, allv)
    plsc.subcore_barrier()

    def mx2(k, m):
        return jnp.maximum(m, allv[pl.ds(k * _L, _L)])

    m16 = jnp.broadcast_to(jnp.max(lax.fori_loop(0, _NW, mx2, neg_inf)), (_L,))

    def exp_pass(nch, unroll):
        def body(k, acc):
            es = []
            for j in range(unroll):
                off = (k * unroll + j) * _L
                e = jnp.exp(buf[pl.ds(off, _L)] - m16)
                buf[pl.ds(off, _L)] = e
                es.append(e)
            return acc + _tree(es, lambda x, y: x + y)

        stage_v[...] = lax.fori_loop(
            0, nch // unroll, body, jnp.zeros((_L,), jnp.float32))

    @pl.when(s < 15)
    def _():
        exp_pass(_CHUNK // _L, 8)

    @pl.when(s == 15)
    def _():
        exp_pass(_LAST // _L, 2)

    pltpu.sync_copy(stage_v, shared.at[pl.ds(s * _L, _L)])
    plsc.subcore_barrier()
    pltpu.sync_copy(shared, allv)

    def sm2(k, acc):
        return acc + allv[pl.ds(k * _L, _L)]

    total = jnp.sum(lax.fori_loop(0, _NW, sm2, jnp.zeros((_L,), jnp.float32)))
    scale16 = (1.0 - a16) / jnp.broadcast_to(total, (_L,))

    def scale_pass(nch, unroll):
        def body(k, carry):
            for j in range(unroll):
                off = (k * unroll + j) * _L
                buf[pl.ds(off, _L)] = buf[pl.ds(off, _L)] * scale16
            return carry

        lax.fori_loop(0, nch // unroll, body, 0)

    @pl.when(s < 15)
    def _():
        scale_pass(_CHUNK // _L, 8)
        pltpu.sync_copy(buf, p_hbm.at[pl.ds(base, _CHUNK)])

    @pl.when(s == 15)
    def _():
        scale_pass(_LAST // _L, 2)
        pltpu.sync_copy(buf.at[pl.ds(0, _LAST)], p_hbm.at[pl.ds(base, _LAST)])


def _sc_softmax(alf16, gf_flat):
    mesh = plsc.VectorSubcoreMesh(
        core_axis_name="c", subcore_axis_name="s", num_cores=1)
    kern = pl.kernel(
        _sc_softmax_body,
        out_type=jax.ShapeDtypeStruct((_N,), jnp.float32),
        mesh=mesh,
        scratch_types=[
            pltpu.VMEM((_CHUNK,), jnp.float32),
            pltpu.VMEM((_L,), jnp.float32),
            pltpu.VMEM((_NW * _L,), jnp.float32),
            pltpu.VMEM_SHARED((_NW * _L,), jnp.float32),
            pltpu.SemaphoreType.DMA,
        ],
        compiler_params=pltpu.CompilerParams(needs_layout_passes=False),
    )
    return kern(alf16, gf_flat)


def _blend_kernel(alf_ref, p_ref, his_ref, out_ref):
    a = jax.nn.sigmoid(alf_ref[0])
    g_col = p_ref[0, 0, :].reshape(_IT, 1)
    out_ref[...] = a * his_ref[...] + g_col


def kernel(his, global_freq_table, alf):
    batch, num_items = his.shape
    alf16 = jnp.broadcast_to(alf, (_L,))
    p = _sc_softmax(alf16, global_freq_table.reshape(-1))

    num_tiles = num_items // _IT
    p3 = p.reshape(num_tiles, 1, _IT)
    his_t = his.T
    out_t = pl.pallas_call(
        _blend_kernel,
        grid=(num_tiles,),
        in_specs=[
            pl.BlockSpec(memory_space=pltpu.SMEM),
            pl.BlockSpec((1, 1, _IT), lambda i: (i, 0, 0)),
            pl.BlockSpec((_IT, batch), lambda i: (i, 0)),
        ],
        out_specs=pl.BlockSpec((_IT, batch), lambda i: (i, 0)),
        out_shape=jax.ShapeDtypeStruct((num_items, batch), his.dtype),
        compiler_params=pltpu.CompilerParams(
            vmem_limit_bytes=100 * 1024 * 1024),
    )(alf, p3, his_t)
    return out_t.T


# speedup vs baseline: 1.0092x; 1.0092x over previous
import jax
import jax.numpy as jnp
from jax import lax
from jax.experimental import pallas as pl
from jax.experimental.pallas import tpu as pltpu
from jax.experimental.pallas import tpu_sc as plsc

_IT = 4000
_NW = 16
_L = 16
_PAD_N = 100352
_CHUNK = _PAD_N // _NW
_NCH = _CHUNK // _L


def _sc_softmax_body(alf_hbm, gf_hbm, p_hbm, buf, stage_v, allv, shared, sem):
    s = lax.axis_index("s")
    base = s * _CHUNK

    pltpu.make_async_copy(gf_hbm.at[pl.ds(base, _CHUNK)], buf, sem).start()
    pltpu.sync_copy(alf_hbm, stage_v)
    alfv = stage_v[...]
    a16 = 1.0 / (1.0 + jnp.exp(-alfv))
    pltpu.make_async_copy(gf_hbm.at[pl.ds(base, _CHUNK)], buf, sem).wait()

    neg_inf = jnp.full((_L,), -jnp.inf, jnp.float32)
    _U = 8

    def mx(k, m):
        vals = [buf[pl.ds((k * _U + j) * _L, _L)] for j in range(_U)]
        t = [jnp.maximum(vals[2 * j], vals[2 * j + 1]) for j in range(_U // 2)]
        t = [jnp.maximum(t[2 * j], t[2 * j + 1]) for j in range(_U // 4)]
        return jnp.maximum(m, jnp.maximum(t[0], t[1]))

    mloc = lax.fori_loop(0, _NCH // _U, mx, neg_inf)
    stage_v[...] = mloc
    pltpu.sync_copy(stage_v, shared.at[pl.ds(s * _L, _L)])
    plsc.subcore_barrier()
    pltpu.sync_copy(shared, allv)
    plsc.subcore_barrier()

    def mx2(k, m):
        return jnp.maximum(m, allv[pl.ds(k * _L, _L)])

    m = jnp.max(lax.fori_loop(0, _NW, mx2, neg_inf))
    m16 = jnp.broadcast_to(m, (_L,))

    def ex(k, acc):
        es = []
        for j in range(_U):
            e = jnp.exp(buf[pl.ds((k * _U + j) * _L, _L)] - m16)
            buf[pl.ds((k * _U + j) * _L, _L)] = e
            es.append(e)
        t = [es[2 * j] + es[2 * j + 1] for j in range(_U // 2)]
        t = [t[2 * j] + t[2 * j + 1] for j in range(_U // 4)]
        return acc + (t[0] + t[1])

    sloc = lax.fori_loop(0, _NCH // _U, ex, jnp.zeros((_L,), jnp.float32))
    stage_v[...] = sloc
    pltpu.sync_copy(stage_v, shared.at[pl.ds(s * _L, _L)])
    plsc.subcore_barrier()
    pltpu.sync_copy(shared, allv)

    def sm2(k, acc):
        return acc + allv[pl.ds(k * _L, _L)]

    total = jnp.sum(lax.fori_loop(0, _NW, sm2, jnp.zeros((_L,), jnp.float32)))
    scale16 = (1.0 - a16) / jnp.broadcast_to(total, (_L,))

    def sc(k, carry):
        for j in range(_U):
            off = (k * _U + j) * _L
            buf[pl.ds(off, _L)] = buf[pl.ds(off, _L)] * scale16
        return carry

    lax.fori_loop(0, _NCH // _U, sc, 0)
    pltpu.sync_copy(buf, p_hbm.at[pl.ds(base, _CHUNK)])


def _sc_softmax(alf16, gf_pad):
    mesh = plsc.VectorSubcoreMesh(
        core_axis_name="c", subcore_axis_name="s", num_cores=1)
    kern = pl.kernel(
        _sc_softmax_body,
        out_type=jax.ShapeDtypeStruct((_PAD_N,), jnp.float32),
        mesh=mesh,
        scratch_types=[
            pltpu.VMEM((_CHUNK,), jnp.float32),
            pltpu.VMEM((_L,), jnp.float32),
            pltpu.VMEM((_NW * _L,), jnp.float32),
            pltpu.VMEM_SHARED((_NW * _L,), jnp.float32),
            pltpu.SemaphoreType.DMA,
        ],
        compiler_params=pltpu.CompilerParams(needs_layout_passes=False),
    )
    return kern(alf16, gf_pad)


def _blend_kernel(alf_ref, p_ref, his_ref, out_ref):
    a = jax.nn.sigmoid(alf_ref[0])
    g_col = p_ref[0, 0, :].reshape(_IT, 1)
    out_ref[...] = a * his_ref[...] + g_col


def kernel(his, global_freq_table, alf):
    batch, num_items = his.shape
    gf_pad = jnp.concatenate(
        [global_freq_table.reshape(-1),
         jnp.full((_PAD_N - num_items,), -jnp.inf, jnp.float32)])
    alf16 = jnp.broadcast_to(alf, (_L,))
    p_pad = _sc_softmax(alf16, gf_pad)

    num_tiles = num_items // _IT
    p3 = p_pad[:num_items].reshape(num_tiles, 1, _IT)
    his_t = his.T
    out_t = pl.pallas_call(
        _blend_kernel,
        grid=(num_tiles,),
        in_specs=[
            pl.BlockSpec(memory_space=pltpu.SMEM),
            pl.BlockSpec((1, 1, _IT), lambda i: (i, 0, 0)),
            pl.BlockSpec((_IT, batch), lambda i: (i, 0)),
        ],
        out_specs=pl.BlockSpec((_IT, batch), lambda i: (i, 0)),
        out_shape=jax.ShapeDtypeStruct((num_items, batch), his.dtype),
        compiler_params=pltpu.CompilerParams(
            vmem_limit_bytes=100 * 1024 * 1024),
    )(alf, p3, his_t)
    return out_t.T
